# Initial kernel scaffold; baseline (speedup 1.0000x reference)
import jax
import jax.numpy as jnp
from jax.experimental import pallas as pl


def kernel(x, hop_edge_index, hop_edge_att, W1, b1, W2, b2, group_weights, temp):
    raise NotImplementedError("write your pallas kernel here")


# speedup vs baseline: 63.4401x; 63.4401x over previous
import functools

import jax
import jax.numpy as jnp
from jax import lax
from jax.experimental import pallas as pl
from jax.experimental.pallas import tpu as pltpu
from jax.experimental.pallas import tpu_sc as plsc

N = 100000
E = 3200000
HOPS = 3
HID = 64
GROUPS = [(0, 16), (16, 32), (32, 48), (48, 58)]

RB = 12544
NA = 8 * RB
DIN = 64

NC, NS, L = 2, 16, 16
NW = NC * NS
NPAD = 100096
SL = NPAD // NS
CR = 8
ROWS_PER_HOP = E // 128
CHUNKS_PER_HOP = ROWS_PER_HOP // CR
KMAX = (CHUNKS_PER_HOP + NW - 1) // NW
ZB = 2048


def _mlp_body(x_ref, gw_ref, w1_ref, b1_ref, w2_ref, b2_ref, o_ref):
    xw = x_ref[...] * gw_ref[...]
    h1 = jnp.maximum(jnp.dot(xw, w1_ref[...], preferred_element_type=jnp.float32)
                     + b1_ref[...], 0.0)
    o_ref[...] = jnp.dot(h1, w2_ref[...], preferred_element_type=jnp.float32) \
        + b2_ref[...]


def _combine_body(a_ref, b_ref, c_ref, o_ref):
    o_ref[...] = a_ref[...] + b_ref[...] + c_ref[...]


def _sc_body(h_hbm, src_hbm, dst_hbm, att_hbm, temp_hbm, out0_hbm, out1_hbm,
             src_v, dst_v, att_v, gat_v, zero_v, temp_v, flush_v, acc_sh,
             sem_g, sem_s):
    cid = lax.axis_index("c")
    sid = lax.axis_index("s")
    g = cid * NS + sid

    def _z(i, _):
        zero_v[pl.ds(i * L, L)] = jnp.zeros((L,), jnp.float32)
        return 0
    lax.fori_loop(0, ZB // L, _z, 0)
    base = sid * SL
    pltpu.sync_copy(zero_v, acc_sh.at[pl.ds(base, ZB)])
    pltpu.sync_copy(zero_v, acc_sh.at[pl.ds(base + ZB, ZB)])
    pltpu.sync_copy(zero_v, acc_sh.at[pl.ds(base + 2 * ZB, ZB)])
    pltpu.sync_copy(zero_v.at[pl.ds(0, SL - 3 * ZB)],
                    acc_sh.at[pl.ds(base + 3 * ZB, SL - 3 * ZB)])
    pltpu.sync_copy(temp_hbm, temp_v)
    plsc.subcore_barrier()

    for hop in range(HOPS):
        t16 = temp_v[hop]

        def _chunk(k, _, hop=hop, t16=t16):
            t = k * NW + g

            @pl.when(t < CHUNKS_PER_HOP)
            def _():
                row = hop * ROWS_PER_HOP + t * CR
                pltpu.sync_copy(src_hbm.at[pl.ds(row, CR), :], src_v)
                pltpu.sync_copy(dst_hbm.at[pl.ds(row, CR), :], dst_v)
                pltpu.sync_copy(att_hbm.at[pl.ds(row, CR), :], att_v)
                gets = [pltpu.async_copy(h_hbm.at[src_v.at[j]], gat_v.at[j],
                                         sem_g) for j in range(CR)]
                for d in gets:
                    d.wait()

                def _mul(i, _):
                    r = i // 8
                    c = (i % 8) * L
                    gat_v[r, pl.ds(c, L)] = (gat_v[r, pl.ds(c, L)]
                                             * att_v[r, pl.ds(c, L)] * t16)
                    return 0
                lax.fori_loop(0, CR * 8, _mul, 0)
                puts = [pltpu.async_copy(gat_v.at[j], acc_sh.at[dst_v.at[j]],
                                         sem_s, add=True) for j in range(CR)]
                for d in puts:
                    d.wait()
            return 0

        lax.fori_loop(0, KMAX, _chunk, 0)

    plsc.subcore_barrier()

    pltpu.sync_copy(acc_sh.at[pl.ds(base, SL)], flush_v)

    @pl.when(cid == 0)
    def _():
        pltpu.sync_copy(flush_v, out0_hbm.at[pl.ds(base, SL)])

    @pl.when(cid == 1)
    def _():
        pltpu.sync_copy(flush_v, out1_hbm.at[pl.ds(base, SL)])


_sc_call = functools.partial(
    pl.kernel,
    mesh=plsc.VectorSubcoreMesh(core_axis_name="c", subcore_axis_name="s"),
    out_type=(jax.ShapeDtypeStruct((NPAD,), jnp.float32),
              jax.ShapeDtypeStruct((NPAD,), jnp.float32)),
    scratch_types=[
        pltpu.VMEM((CR, 128), jnp.int32),
        pltpu.VMEM((CR, 128), jnp.int32),
        pltpu.VMEM((CR, 128), jnp.float32),
        pltpu.VMEM((CR, 128), jnp.float32),
        pltpu.VMEM((ZB,), jnp.float32),
        pltpu.VMEM((HOPS, L), jnp.float32),
        pltpu.VMEM((SL,), jnp.float32),
        pltpu.VMEM_SHARED((NPAD,), jnp.float32),
        pltpu.SemaphoreType.DMA,
        pltpu.SemaphoreType.DMA,
    ],
)(_sc_body)


def kernel(x, hop_edge_index, hop_edge_att, W1, b1, W2, b2, group_weights, temp):
    f32 = jnp.float32
    gw = jnp.concatenate(
        [jnp.full((e - s,), 1.0, f32) * group_weights[i]
         for i, (s, e) in enumerate(GROUPS)]
        + [jnp.zeros((DIN - 58,), f32)])

    x_pad = jnp.zeros((NA, DIN), f32).at[:N, :58].set(x)
    w1_pad = jnp.zeros((DIN, HID), f32).at[:58, :].set(W1)

    h_full = pl.pallas_call(
        _mlp_body,
        grid=(NA // RB,),
        in_specs=[
            pl.BlockSpec((RB, DIN), lambda i: (i, 0)),
            pl.BlockSpec((1, DIN), lambda i: (0, 0)),
            pl.BlockSpec((DIN, HID), lambda i: (0, 0)),
            pl.BlockSpec((1, HID), lambda i: (0, 0)),
            pl.BlockSpec((HID, 1), lambda i: (0, 0)),
            pl.BlockSpec((1, 1), lambda i: (0, 0)),
        ],
        out_specs=pl.BlockSpec((RB, 1), lambda i: (i, 0)),
        out_shape=jax.ShapeDtypeStruct((NA, 1), f32),
    )(x_pad, gw[None, :], w1_pad, b1[None, :], W2, b2[None, :])

    h_flat = h_full.reshape(NA)[:NPAD]

    src3 = hop_edge_index[:, 0, :].reshape(HOPS * ROWS_PER_HOP, 128)
    dst3 = hop_edge_index[:, 1, :].reshape(HOPS * ROWS_PER_HOP, 128)
    att3 = hop_edge_att.reshape(HOPS * ROWS_PER_HOP, 128)
    temp_b = jnp.broadcast_to(temp[:, None], (HOPS, L))

    p0, p1 = _sc_call(h_flat, src3, dst3, att3, temp_b)

    out2d = pl.pallas_call(
        _combine_body,
        out_shape=jax.ShapeDtypeStruct((NPAD // 128, 128), f32),
    )(p0.reshape(NPAD // 128, 128),
      p1.reshape(NPAD // 128, 128),
      h_flat.reshape(NPAD // 128, 128))

    return out2d.reshape(NPAD)[:N].reshape(N, 1)


# speedup vs baseline: 112.3444x; 1.7709x over previous
import functools

import jax
import jax.numpy as jnp
from jax import lax
from jax.experimental import pallas as pl
from jax.experimental.pallas import tpu as pltpu
from jax.experimental.pallas import tpu_sc as plsc

N = 100000
E = 3200000
HOPS = 3
HID = 64
GROUPS = [(0, 16), (16, 32), (32, 48), (48, 58)]

RB = 12544
NA = 8 * RB
DIN = 64

NC, NS, L = 2, 16, 16
NW = NC * NS
NPAD = 100096
SL = NPAD // NS
CR = 8
ROWS_PER_HOP = E // 128
CHUNKS_PER_HOP = ROWS_PER_HOP // CR
KMAX = (CHUNKS_PER_HOP + NW - 1) // NW
ZB = 2048


def _mlp_body(x_ref, gw_ref, w1_ref, b1_ref, w2_ref, b2_ref, o_ref):
    xw = x_ref[...] * gw_ref[...]
    h1 = jnp.maximum(jnp.dot(xw, w1_ref[...], preferred_element_type=jnp.float32)
                     + b1_ref[...], 0.0)
    o_ref[...] = jnp.dot(h1, w2_ref[...], preferred_element_type=jnp.float32) \
        + b2_ref[...]


def _combine_body(a_ref, b_ref, c_ref, o_ref):
    o_ref[...] = a_ref[...] + b_ref[...] + c_ref[...]


def _sc_body(h_hbm, src_hbm, dst_hbm, att_hbm, temp_hbm, out0_hbm, out1_hbm,
             src_v, dst_v, att_v, gat_v, zero_v, temp_v, flush_v, acc_sh, h_sh,
             sem_g, sem_s, sem_l):
    cid = lax.axis_index("c")
    sid = lax.axis_index("s")
    g = cid * NS + sid
    base = sid * SL

    pltpu.sync_copy(h_hbm.at[pl.ds(base, SL)], flush_v)
    pltpu.sync_copy(flush_v, h_sh.at[pl.ds(base, SL)])

    def _z(i, _):
        zero_v[pl.ds(i * L, L)] = jnp.zeros((L,), jnp.float32)
        return 0
    lax.fori_loop(0, ZB // L, _z, 0)
    pltpu.sync_copy(zero_v, acc_sh.at[pl.ds(base, ZB)])
    pltpu.sync_copy(zero_v, acc_sh.at[pl.ds(base + ZB, ZB)])
    pltpu.sync_copy(zero_v, acc_sh.at[pl.ds(base + 2 * ZB, ZB)])
    pltpu.sync_copy(zero_v.at[pl.ds(0, SL - 3 * ZB)],
                    acc_sh.at[pl.ds(base + 3 * ZB, SL - 3 * ZB)])
    pltpu.sync_copy(temp_hbm, temp_v)
    plsc.subcore_barrier()

    for hop in range(HOPS):
        t16 = temp_v[hop]

        def _chunk(k, _, hop=hop, t16=t16):
            t = k * NW + g

            @pl.when(t < CHUNKS_PER_HOP)
            def _():
                row = hop * ROWS_PER_HOP + t * CR
                loads = [
                    pltpu.async_copy(src_hbm.at[pl.ds(row, CR), :], src_v, sem_l),
                    pltpu.async_copy(dst_hbm.at[pl.ds(row, CR), :], dst_v, sem_l),
                    pltpu.async_copy(att_hbm.at[pl.ds(row, CR), :], att_v, sem_l),
                ]
                for d in loads:
                    d.wait()
                gets = [pltpu.async_copy(h_sh.at[src_v.at[j]], gat_v.at[j],
                                         sem_g) for j in range(CR)]
                for d in gets:
                    d.wait()

                def _mul(i, _):
                    r = i // 8
                    c = (i % 8) * L
                    gat_v[r, pl.ds(c, L)] = (gat_v[r, pl.ds(c, L)]
                                             * att_v[r, pl.ds(c, L)] * t16)
                    return 0
                lax.fori_loop(0, CR * 8, _mul, 0)
                puts = [pltpu.async_copy(gat_v.at[j], acc_sh.at[dst_v.at[j]],
                                         sem_s, add=True) for j in range(CR)]
                for d in puts:
                    d.wait()
            return 0

        lax.fori_loop(0, KMAX, _chunk, 0)

    plsc.subcore_barrier()

    pltpu.sync_copy(acc_sh.at[pl.ds(base, SL)], flush_v)

    @pl.when(cid == 0)
    def _():
        pltpu.sync_copy(flush_v, out0_hbm.at[pl.ds(base, SL)])

    @pl.when(cid == 1)
    def _():
        pltpu.sync_copy(flush_v, out1_hbm.at[pl.ds(base, SL)])


_sc_call = functools.partial(
    pl.kernel,
    mesh=plsc.VectorSubcoreMesh(core_axis_name="c", subcore_axis_name="s"),
    out_type=(jax.ShapeDtypeStruct((NPAD,), jnp.float32),
              jax.ShapeDtypeStruct((NPAD,), jnp.float32)),
    scratch_types=[
        pltpu.VMEM((CR, 128), jnp.int32),
        pltpu.VMEM((CR, 128), jnp.int32),
        pltpu.VMEM((CR, 128), jnp.float32),
        pltpu.VMEM((CR, 128), jnp.float32),
        pltpu.VMEM((ZB,), jnp.float32),
        pltpu.VMEM((HOPS, L), jnp.float32),
        pltpu.VMEM((SL,), jnp.float32),
        pltpu.VMEM_SHARED((NPAD,), jnp.float32),
        pltpu.VMEM_SHARED((NPAD,), jnp.float32),
        pltpu.SemaphoreType.DMA,
        pltpu.SemaphoreType.DMA,
        pltpu.SemaphoreType.DMA,
    ],
)(_sc_body)


def kernel(x, hop_edge_index, hop_edge_att, W1, b1, W2, b2, group_weights, temp):
    f32 = jnp.float32
    gw = jnp.concatenate(
        [jnp.full((e - s,), 1.0, f32) * group_weights[i]
         for i, (s, e) in enumerate(GROUPS)]
        + [jnp.zeros((DIN - 58,), f32)])

    x_pad = jnp.zeros((NA, DIN), f32).at[:N, :58].set(x)
    w1_pad = jnp.zeros((DIN, HID), f32).at[:58, :].set(W1)

    h_full = pl.pallas_call(
        _mlp_body,
        grid=(NA // RB,),
        in_specs=[
            pl.BlockSpec((RB, DIN), lambda i: (i, 0)),
            pl.BlockSpec((1, DIN), lambda i: (0, 0)),
            pl.BlockSpec((DIN, HID), lambda i: (0, 0)),
            pl.BlockSpec((1, HID), lambda i: (0, 0)),
            pl.BlockSpec((HID, 1), lambda i: (0, 0)),
            pl.BlockSpec((1, 1), lambda i: (0, 0)),
        ],
        out_specs=pl.BlockSpec((RB, 1), lambda i: (i, 0)),
        out_shape=jax.ShapeDtypeStruct((NA, 1), f32),
    )(x_pad, gw[None, :], w1_pad, b1[None, :], W2, b2[None, :])

    h_flat = h_full.reshape(NA)[:NPAD]

    src3 = hop_edge_index[:, 0, :].reshape(HOPS * ROWS_PER_HOP, 128)
    dst3 = hop_edge_index[:, 1, :].reshape(HOPS * ROWS_PER_HOP, 128)
    att3 = hop_edge_att.reshape(HOPS * ROWS_PER_HOP, 128)
    temp_b = jnp.broadcast_to(temp[:, None], (HOPS, L))

    p0, p1 = _sc_call(h_flat, src3, dst3, att3, temp_b)

    out2d = pl.pallas_call(
        _combine_body,
        out_shape=jax.ShapeDtypeStruct((NPAD // 128, 128), f32),
    )(p0.reshape(NPAD // 128, 128),
      p1.reshape(NPAD // 128, 128),
      h_flat.reshape(NPAD // 128, 128))

    return out2d.reshape(NPAD)[:N].reshape(N, 1)


# speedup vs baseline: 122.7363x; 1.0925x over previous
import functools

import jax
import jax.numpy as jnp
from jax import lax
from jax.experimental import pallas as pl
from jax.experimental.pallas import tpu as pltpu
from jax.experimental.pallas import tpu_sc as plsc

N = 100000
E = 3200000
HOPS = 3
HID = 64
GROUPS = [(0, 16), (16, 32), (32, 48), (48, 58)]

RB = 12544
NA = 8 * RB
DIN = 64

NC, NS, L = 2, 16, 16
NW = NC * NS
NPAD = 100096
SL = NPAD // NS
CR = 8
ROWS_PER_HOP = E // 128
CHUNKS_PER_HOP = ROWS_PER_HOP // CR
KMAX = (CHUNKS_PER_HOP + NW - 1) // NW
ZB = 2048


def _mlp_body(x_ref, gw_ref, w1_ref, b1_ref, w2_ref, b2_ref, o_ref):
    xw = x_ref[...] * gw_ref[...]
    h1 = jnp.maximum(jnp.dot(xw, w1_ref[...], preferred_element_type=jnp.float32)
                     + b1_ref[...], 0.0)
    o_ref[...] = jnp.dot(h1, w2_ref[...], preferred_element_type=jnp.float32) \
        + b2_ref[...]


def _combine_body(a_ref, b_ref, c_ref, o_ref):
    o_ref[...] = a_ref[...] + b_ref[...] + c_ref[...]


def _sc_body(h_hbm, src_hbm, dst_hbm, att_hbm, temp_hbm, out0_hbm, out1_hbm,
             src_v, dst_v, att_v, gat_v, zero_v, temp_v, flush_v, acc_sh, h_sh,
             sem_g, sem_s, sem_l):
    cid = lax.axis_index("c")
    sid = lax.axis_index("s")
    g = cid * NS + sid
    base = sid * SL

    pltpu.sync_copy(h_hbm.at[pl.ds(base, SL)], flush_v)
    pltpu.sync_copy(flush_v, h_sh.at[pl.ds(base, SL)])

    def _z(i, _):
        zero_v[pl.ds(i * L, L)] = jnp.zeros((L,), jnp.float32)
        return 0
    lax.fori_loop(0, ZB // L, _z, 0)
    pltpu.sync_copy(zero_v, acc_sh.at[pl.ds(base, ZB)])
    pltpu.sync_copy(zero_v, acc_sh.at[pl.ds(base + ZB, ZB)])
    pltpu.sync_copy(zero_v, acc_sh.at[pl.ds(base + 2 * ZB, ZB)])
    pltpu.sync_copy(zero_v.at[pl.ds(0, SL - 3 * ZB)],
                    acc_sh.at[pl.ds(base + 3 * ZB, SL - 3 * ZB)])
    pltpu.sync_copy(temp_hbm, temp_v)
    plsc.subcore_barrier()

    for hop in range(HOPS):
        t16 = temp_v[hop]

        def _chunk(k, _, hop=hop, t16=t16):
            t = k * NW + g

            @pl.when(t < CHUNKS_PER_HOP)
            def _():
                row = hop * ROWS_PER_HOP + t * CR
                loads = [
                    pltpu.async_copy(src_hbm.at[pl.ds(row, CR), :], src_v, sem_l),
                    pltpu.async_copy(dst_hbm.at[pl.ds(row, CR), :], dst_v, sem_l),
                    pltpu.async_copy(att_hbm.at[pl.ds(row, CR), :], att_v, sem_l),
                ]
                for d in loads:
                    d.wait()
                gets = [pltpu.async_copy(h_sh.at[src_v.at[j]], gat_v.at[j],
                                         sem_g) for j in range(CR)]
                for d in gets:
                    d.wait()

                def _mul(r, _):
                    for u in range(8):
                        c = u * L
                        gat_v[r, pl.ds(c, L)] = (gat_v[r, pl.ds(c, L)]
                                                 * att_v[r, pl.ds(c, L)] * t16)
                    return 0
                lax.fori_loop(0, CR, _mul, 0)
                puts = [pltpu.async_copy(gat_v.at[j], acc_sh.at[dst_v.at[j]],
                                         sem_s, add=True) for j in range(CR)]
                for d in puts:
                    d.wait()
            return 0

        lax.fori_loop(0, KMAX, _chunk, 0)

    plsc.subcore_barrier()

    pltpu.sync_copy(acc_sh.at[pl.ds(base, SL)], flush_v)

    @pl.when(cid == 0)
    def _():
        pltpu.sync_copy(flush_v, out0_hbm.at[pl.ds(base, SL)])

    @pl.when(cid == 1)
    def _():
        pltpu.sync_copy(flush_v, out1_hbm.at[pl.ds(base, SL)])


_sc_call = functools.partial(
    pl.kernel,
    mesh=plsc.VectorSubcoreMesh(core_axis_name="c", subcore_axis_name="s"),
    out_type=(jax.ShapeDtypeStruct((NPAD,), jnp.float32),
              jax.ShapeDtypeStruct((NPAD,), jnp.float32)),
    scratch_types=[
        pltpu.VMEM((CR, 128), jnp.int32),
        pltpu.VMEM((CR, 128), jnp.int32),
        pltpu.VMEM((CR, 128), jnp.float32),
        pltpu.VMEM((CR, 128), jnp.float32),
        pltpu.VMEM((ZB,), jnp.float32),
        pltpu.VMEM((HOPS, L), jnp.float32),
        pltpu.VMEM((SL,), jnp.float32),
        pltpu.VMEM_SHARED((NPAD,), jnp.float32),
        pltpu.VMEM_SHARED((NPAD,), jnp.float32),
        pltpu.SemaphoreType.DMA,
        pltpu.SemaphoreType.DMA,
        pltpu.SemaphoreType.DMA,
    ],
)(_sc_body)


def kernel(x, hop_edge_index, hop_edge_att, W1, b1, W2, b2, group_weights, temp):
    f32 = jnp.float32
    gw = jnp.concatenate(
        [jnp.full((e - s,), 1.0, f32) * group_weights[i]
         for i, (s, e) in enumerate(GROUPS)]
        + [jnp.zeros((DIN - 58,), f32)])

    x_pad = jnp.zeros((NA, DIN), f32).at[:N, :58].set(x)
    w1_pad = jnp.zeros((DIN, HID), f32).at[:58, :].set(W1)

    h_full = pl.pallas_call(
        _mlp_body,
        grid=(NA // RB,),
        in_specs=[
            pl.BlockSpec((RB, DIN), lambda i: (i, 0)),
            pl.BlockSpec((1, DIN), lambda i: (0, 0)),
            pl.BlockSpec((DIN, HID), lambda i: (0, 0)),
            pl.BlockSpec((1, HID), lambda i: (0, 0)),
            pl.BlockSpec((HID, 1), lambda i: (0, 0)),
            pl.BlockSpec((1, 1), lambda i: (0, 0)),
        ],
        out_specs=pl.BlockSpec((RB, 1), lambda i: (i, 0)),
        out_shape=jax.ShapeDtypeStruct((NA, 1), f32),
    )(x_pad, gw[None, :], w1_pad, b1[None, :], W2, b2[None, :])

    h_flat = h_full.reshape(NA)[:NPAD]

    src3 = hop_edge_index[:, 0, :].reshape(HOPS * ROWS_PER_HOP, 128)
    dst3 = hop_edge_index[:, 1, :].reshape(HOPS * ROWS_PER_HOP, 128)
    att3 = hop_edge_att.reshape(HOPS * ROWS_PER_HOP, 128)
    temp_b = jnp.broadcast_to(temp[:, None], (HOPS, L))

    p0, p1 = _sc_call(h_flat, src3, dst3, att3, temp_b)

    out2d = pl.pallas_call(
        _combine_body,
        out_shape=jax.ShapeDtypeStruct((NPAD // 128, 128), f32),
    )(p0.reshape(NPAD // 128, 128),
      p1.reshape(NPAD // 128, 128),
      h_flat.reshape(NPAD // 128, 128))

    return out2d.reshape(NPAD)[:N].reshape(N, 1)


# speedup vs baseline: 181.2870x; 1.4770x over previous
import functools

import jax
import jax.numpy as jnp
from jax import lax
from jax.experimental import pallas as pl
from jax.experimental.pallas import tpu as pltpu
from jax.experimental.pallas import tpu_sc as plsc

N = 100000
E = 3200000
HOPS = 3
HID = 64
GROUPS = [(0, 16), (16, 32), (32, 48), (48, 58)]

RB = 12544
NA = 8 * RB
DIN = 64

NC, NS, L = 2, 16, 16
NW = NC * NS
NPAD = 100096
SL = NPAD // NS
CR = 8
ROWS_PER_HOP = E // 128
CHUNKS_PER_HOP = ROWS_PER_HOP // CR
TOTAL_CHUNKS = HOPS * CHUNKS_PER_HOP
CPW = (TOTAL_CHUNKS + NW - 1) // NW
TMAX = (CPW + 2) // 3
ZB = 2048


def _mlp_body(x_ref, gw_ref, w1_ref, b1_ref, w2_ref, b2_ref, o_ref):
    xw = x_ref[...] * gw_ref[...]
    h1 = jnp.maximum(jnp.dot(xw, w1_ref[...], preferred_element_type=jnp.float32)
                     + b1_ref[...], 0.0)
    o_ref[...] = jnp.dot(h1, w2_ref[...], preferred_element_type=jnp.float32) \
        + b2_ref[...]


def _combine_body(a_ref, b_ref, c_ref, o_ref):
    o_ref[...] = a_ref[...] + b_ref[...] + c_ref[...]


def _sc_body(h_hbm, src_hbm, dst_hbm, att_hbm, temp_hbm, out0_hbm, out1_hbm,
             *sc):
    (src0, dst0, att0, gat0, src1, dst1, att1, gat1, src2, dst2, att2, gat2,
     zero_v, temp_v, flush_v, acc_sh, h_sh,
     sl0, sl1, sl2, ss0, ss1, ss2, sem_g) = sc
    srcs = (src0, src1, src2)
    dsts = (dst0, dst1, dst2)
    atts = (att0, att1, att2)
    gats = (gat0, gat1, gat2)
    sls = (sl0, sl1, sl2)
    sss = (ss0, ss1, ss2)
    cid = lax.axis_index("c")
    sid = lax.axis_index("s")
    g = cid * NS + sid
    base = sid * SL

    pltpu.sync_copy(h_hbm.at[pl.ds(base, SL)], flush_v)
    pltpu.sync_copy(flush_v, h_sh.at[pl.ds(base, SL)])

    def _z(i, _):
        zero_v[pl.ds(i * L, L)] = jnp.zeros((L,), jnp.float32)
        return 0
    lax.fori_loop(0, ZB // L, _z, 0)
    pltpu.sync_copy(zero_v, acc_sh.at[pl.ds(base, ZB)])
    pltpu.sync_copy(zero_v, acc_sh.at[pl.ds(base + ZB, ZB)])
    pltpu.sync_copy(zero_v, acc_sh.at[pl.ds(base + 2 * ZB, ZB)])
    pltpu.sync_copy(zero_v.at[pl.ds(0, SL - 3 * ZB)],
                    acc_sh.at[pl.ds(base + 3 * ZB, SL - 3 * ZB)])
    pltpu.sync_copy(temp_hbm, temp_v)
    plsc.subcore_barrier()

    q0 = g * CPW
    cnt = jnp.minimum(CPW, TOTAL_CHUNKS - q0)

    def fire_loads(c, s):
        row = (q0 + c) * CR
        pltpu.async_copy(src_hbm.at[pl.ds(row, CR), :], srcs[s], sls[s])
        pltpu.async_copy(dst_hbm.at[pl.ds(row, CR), :], dsts[s], sls[s])
        pltpu.async_copy(att_hbm.at[pl.ds(row, CR), :], atts[s], sls[s])

    def wait_loads(s):
        pltpu.make_async_copy(src_hbm.at[pl.ds(0, CR), :], srcs[s], sls[s]).wait()
        pltpu.make_async_copy(dst_hbm.at[pl.ds(0, CR), :], dsts[s], sls[s]).wait()
        pltpu.make_async_copy(att_hbm.at[pl.ds(0, CR), :], atts[s], sls[s]).wait()

    def drain_scatters(s):
        for j in range(CR):
            pltpu.make_async_copy(gats[s].at[j], acc_sh.at[dsts[s].at[j]],
                                  sss[s]).wait()

    def process(c, s):
        wait_loads(s)
        gets = [pltpu.async_copy(h_sh.at[srcs[s].at[j]], gats[s].at[j], sem_g)
                for j in range(CR)]
        for d in gets:
            d.wait()
        hop = (q0 + c) // CHUNKS_PER_HOP
        t16 = temp_v[hop]

        def _mul(r, _):
            for u in range(8):
                cc = u * L
                gats[s][r, pl.ds(cc, L)] = (gats[s][r, pl.ds(cc, L)]
                                            * atts[s][r, pl.ds(cc, L)] * t16)
            return 0
        lax.fori_loop(0, CR, _mul, 0)
        for j in range(CR):
            pltpu.async_copy(gats[s].at[j], acc_sh.at[dsts[s].at[j]], sss[s],
                             add=True)

    @pl.when(cnt > 0)
    def _():
        fire_loads(0, 0)

    def _triple(t, _):
        for u in range(3):
            c = t * 3 + u

            @pl.when(c < cnt)
            def _(c=c, u=u):
                @pl.when(c >= 2)
                def _():
                    drain_scatters((u + 1) % 3)

                @pl.when(c + 1 < cnt)
                def _():
                    fire_loads(c + 1, (u + 1) % 3)

                process(c, u)
        return 0

    lax.fori_loop(0, TMAX, _triple, 0)

    for s in range(3):
        @pl.when((cnt >= 1) & ((cnt - 1) % 3 == s))
        def _(s=s):
            drain_scatters(s)

        @pl.when((cnt >= 2) & ((cnt - 2) % 3 == s))
        def _(s=s):
            drain_scatters(s)

    plsc.subcore_barrier()

    pltpu.sync_copy(acc_sh.at[pl.ds(base, SL)], flush_v)

    @pl.when(cid == 0)
    def _():
        pltpu.sync_copy(flush_v, out0_hbm.at[pl.ds(base, SL)])

    @pl.when(cid == 1)
    def _():
        pltpu.sync_copy(flush_v, out1_hbm.at[pl.ds(base, SL)])


_sc_call = functools.partial(
    pl.kernel,
    mesh=plsc.VectorSubcoreMesh(core_axis_name="c", subcore_axis_name="s"),
    out_type=(jax.ShapeDtypeStruct((NPAD,), jnp.float32),
              jax.ShapeDtypeStruct((NPAD,), jnp.float32)),
    scratch_types=(
        [pltpu.VMEM((CR, 128), dt)
         for _ in range(3) for dt in (jnp.int32, jnp.int32,
                                      jnp.float32, jnp.float32)]
        + [
            pltpu.VMEM((ZB,), jnp.float32),
            pltpu.VMEM((HOPS, L), jnp.float32),
            pltpu.VMEM((SL,), jnp.float32),
            pltpu.VMEM_SHARED((NPAD,), jnp.float32),
            pltpu.VMEM_SHARED((NPAD,), jnp.float32),
        ]
        + [pltpu.SemaphoreType.DMA] * 7
    ),
)(_sc_body)


def kernel(x, hop_edge_index, hop_edge_att, W1, b1, W2, b2, group_weights, temp):
    f32 = jnp.float32
    gw = jnp.concatenate(
        [jnp.full((e - s,), 1.0, f32) * group_weights[i]
         for i, (s, e) in enumerate(GROUPS)]
        + [jnp.zeros((DIN - 58,), f32)])

    x_pad = jnp.zeros((NA, DIN), f32).at[:N, :58].set(x)
    w1_pad = jnp.zeros((DIN, HID), f32).at[:58, :].set(W1)

    h_full = pl.pallas_call(
        _mlp_body,
        grid=(NA // RB,),
        in_specs=[
            pl.BlockSpec((RB, DIN), lambda i: (i, 0)),
            pl.BlockSpec((1, DIN), lambda i: (0, 0)),
            pl.BlockSpec((DIN, HID), lambda i: (0, 0)),
            pl.BlockSpec((1, HID), lambda i: (0, 0)),
            pl.BlockSpec((HID, 1), lambda i: (0, 0)),
            pl.BlockSpec((1, 1), lambda i: (0, 0)),
        ],
        out_specs=pl.BlockSpec((RB, 1), lambda i: (i, 0)),
        out_shape=jax.ShapeDtypeStruct((NA, 1), f32),
    )(x_pad, gw[None, :], w1_pad, b1[None, :], W2, b2[None, :])

    h_flat = h_full.reshape(NA)[:NPAD]

    src3 = hop_edge_index[:, 0, :].reshape(HOPS * ROWS_PER_HOP, 128)
    dst3 = hop_edge_index[:, 1, :].reshape(HOPS * ROWS_PER_HOP, 128)
    att3 = hop_edge_att.reshape(HOPS * ROWS_PER_HOP, 128)
    temp_b = jnp.broadcast_to(temp[:, None], (HOPS, L))

    p0, p1 = _sc_call(h_flat, src3, dst3, att3, temp_b)

    out2d = pl.pallas_call(
        _combine_body,
        out_shape=jax.ShapeDtypeStruct((NPAD // 128, 128), f32),
    )(p0.reshape(NPAD // 128, 128),
      p1.reshape(NPAD // 128, 128),
      h_flat.reshape(NPAD // 128, 128))

    return out2d.reshape(NPAD)[:N].reshape(N, 1)


# speedup vs baseline: 199.4169x; 1.1000x over previous
import functools

import jax
import jax.numpy as jnp
from jax import lax
from jax.experimental import pallas as pl
from jax.experimental.pallas import tpu as pltpu
from jax.experimental.pallas import tpu_sc as plsc

N = 100000
E = 3200000
HOPS = 3
HID = 64
GROUPS = [(0, 16), (16, 32), (32, 48), (48, 58)]

RB = 12544
NA = 8 * RB
DIN = 64

NC, NS, L = 2, 16, 16
NW = NC * NS
NPAD = 100096
SL = NPAD // NS
CR = 8
ROWS_PER_HOP = E // 128
CHUNKS_PER_HOP = ROWS_PER_HOP // CR
TOTAL_CHUNKS = HOPS * CHUNKS_PER_HOP
CPW = (TOTAL_CHUNKS + NW - 1) // NW
TMAX = (CPW + 2) // 3
ZB = 2048


def _mlp_body(x_ref, gw_ref, w1_ref, b1_ref, w2_ref, b2_ref, o_ref):
    xw = x_ref[...] * gw_ref[...]
    h1 = jnp.maximum(jnp.dot(xw, w1_ref[...], preferred_element_type=jnp.float32)
                     + b1_ref[...], 0.0)
    o_ref[...] = jnp.dot(h1, w2_ref[...], preferred_element_type=jnp.float32) \
        + b2_ref[...]


def _combine_body(a_ref, b_ref, c_ref, o_ref):
    o_ref[...] = a_ref[...] + b_ref[...] + c_ref[...]


def _sc_body(h_hbm, src_hbm, dst_hbm, att_hbm, temp_hbm, out0_hbm, out1_hbm,
             *sc):
    (src0, dst0, att0, gat0, src1, dst1, att1, gat1, src2, dst2, att2, gat2,
     zero_v, temp_v, h_vmem, acc_sh,
     sl0, sl1, sl2, ss0, ss1, ss2) = sc
    srcs = (src0, src1, src2)
    dsts = (dst0, dst1, dst2)
    atts = (att0, att1, att2)
    gats = (gat0, gat1, gat2)
    sls = (sl0, sl1, sl2)
    sss = (ss0, ss1, ss2)
    cid = lax.axis_index("c")
    sid = lax.axis_index("s")
    g = cid * NS + sid
    base = sid * SL

    pltpu.sync_copy(h_hbm, h_vmem)

    def _z(i, _):
        zero_v[pl.ds(i * L, L)] = jnp.zeros((L,), jnp.float32)
        return 0
    lax.fori_loop(0, ZB // L, _z, 0)
    pltpu.sync_copy(zero_v, acc_sh.at[pl.ds(base, ZB)])
    pltpu.sync_copy(zero_v, acc_sh.at[pl.ds(base + ZB, ZB)])
    pltpu.sync_copy(zero_v, acc_sh.at[pl.ds(base + 2 * ZB, ZB)])
    pltpu.sync_copy(zero_v.at[pl.ds(0, SL - 3 * ZB)],
                    acc_sh.at[pl.ds(base + 3 * ZB, SL - 3 * ZB)])
    pltpu.sync_copy(temp_hbm, temp_v)
    plsc.subcore_barrier()

    q0 = g * CPW
    cnt = jnp.minimum(CPW, TOTAL_CHUNKS - q0)

    def fire_loads(c, s):
        row = (q0 + c) * CR
        pltpu.async_copy(src_hbm.at[pl.ds(row, CR), :], srcs[s], sls[s])
        pltpu.async_copy(dst_hbm.at[pl.ds(row, CR), :], dsts[s], sls[s])
        pltpu.async_copy(att_hbm.at[pl.ds(row, CR), :], atts[s], sls[s])

    def wait_loads(s):
        pltpu.make_async_copy(src_hbm.at[pl.ds(0, CR), :], srcs[s], sls[s]).wait()
        pltpu.make_async_copy(dst_hbm.at[pl.ds(0, CR), :], dsts[s], sls[s]).wait()
        pltpu.make_async_copy(att_hbm.at[pl.ds(0, CR), :], atts[s], sls[s]).wait()

    def drain_scatters(s):
        for j in range(CR):
            pltpu.make_async_copy(gats[s].at[j], acc_sh.at[dsts[s].at[j]],
                                  sss[s]).wait()

    def process(c, s):
        wait_loads(s)
        hop = (q0 + c) // CHUNKS_PER_HOP
        t16 = temp_v[hop]

        def _mul(r, _):
            for u in range(8):
                cc = u * L
                idx = srcs[s][r, pl.ds(cc, L)]
                vals = plsc.load_gather(h_vmem, [idx])
                gats[s][r, pl.ds(cc, L)] = vals * atts[s][r, pl.ds(cc, L)] * t16
            return 0
        lax.fori_loop(0, CR, _mul, 0)
        for j in range(CR):
            pltpu.async_copy(gats[s].at[j], acc_sh.at[dsts[s].at[j]], sss[s],
                             add=True)

    @pl.when(cnt > 0)
    def _():
        fire_loads(0, 0)

    def _triple(t, _):
        for u in range(3):
            c = t * 3 + u

            @pl.when(c < cnt)
            def _(c=c, u=u):
                @pl.when(c >= 2)
                def _():
                    drain_scatters((u + 1) % 3)

                @pl.when(c + 1 < cnt)
                def _():
                    fire_loads(c + 1, (u + 1) % 3)

                process(c, u)
        return 0

    lax.fori_loop(0, TMAX, _triple, 0)

    for s in range(3):
        @pl.when((cnt >= 1) & ((cnt - 1) % 3 == s))
        def _(s=s):
            drain_scatters(s)

        @pl.when((cnt >= 2) & ((cnt - 2) % 3 == s))
        def _(s=s):
            drain_scatters(s)

    plsc.subcore_barrier()

    pltpu.sync_copy(acc_sh.at[pl.ds(base, SL)], h_vmem.at[pl.ds(0, SL)])

    @pl.when(cid == 0)
    def _():
        pltpu.sync_copy(h_vmem.at[pl.ds(0, SL)], out0_hbm.at[pl.ds(base, SL)])

    @pl.when(cid == 1)
    def _():
        pltpu.sync_copy(h_vmem.at[pl.ds(0, SL)], out1_hbm.at[pl.ds(base, SL)])


_sc_call = functools.partial(
    pl.kernel,
    mesh=plsc.VectorSubcoreMesh(core_axis_name="c", subcore_axis_name="s"),
    compiler_params=pltpu.CompilerParams(needs_layout_passes=False),
    out_type=(jax.ShapeDtypeStruct((NPAD,), jnp.float32),
              jax.ShapeDtypeStruct((NPAD,), jnp.float32)),
    scratch_types=(
        [pltpu.VMEM((CR, 128), dt)
         for _ in range(3) for dt in (jnp.int32, jnp.int32,
                                      jnp.float32, jnp.float32)]
        + [
            pltpu.VMEM((ZB,), jnp.float32),
            pltpu.VMEM((HOPS, L), jnp.float32),
            pltpu.VMEM((NPAD,), jnp.float32),
            pltpu.VMEM_SHARED((NPAD,), jnp.float32),
        ]
        + [pltpu.SemaphoreType.DMA] * 6
    ),
)(_sc_body)


def kernel(x, hop_edge_index, hop_edge_att, W1, b1, W2, b2, group_weights, temp):
    f32 = jnp.float32
    gw = jnp.concatenate(
        [jnp.full((e - s,), 1.0, f32) * group_weights[i]
         for i, (s, e) in enumerate(GROUPS)]
        + [jnp.zeros((DIN - 58,), f32)])

    x_pad = jnp.zeros((NA, DIN), f32).at[:N, :58].set(x)
    w1_pad = jnp.zeros((DIN, HID), f32).at[:58, :].set(W1)

    h_full = pl.pallas_call(
        _mlp_body,
        grid=(NA // RB,),
        in_specs=[
            pl.BlockSpec((RB, DIN), lambda i: (i, 0)),
            pl.BlockSpec((1, DIN), lambda i: (0, 0)),
            pl.BlockSpec((DIN, HID), lambda i: (0, 0)),
            pl.BlockSpec((1, HID), lambda i: (0, 0)),
            pl.BlockSpec((HID, 1), lambda i: (0, 0)),
            pl.BlockSpec((1, 1), lambda i: (0, 0)),
        ],
        out_specs=pl.BlockSpec((RB, 1), lambda i: (i, 0)),
        out_shape=jax.ShapeDtypeStruct((NA, 1), f32),
    )(x_pad, gw[None, :], w1_pad, b1[None, :], W2, b2[None, :])

    h_flat = h_full.reshape(NA)[:NPAD]

    src3 = hop_edge_index[:, 0, :].reshape(HOPS * ROWS_PER_HOP, 128)
    dst3 = hop_edge_index[:, 1, :].reshape(HOPS * ROWS_PER_HOP, 128)
    att3 = hop_edge_att.reshape(HOPS * ROWS_PER_HOP, 128)
    temp_b = jnp.broadcast_to(temp[:, None], (HOPS, L))

    p0, p1 = _sc_call(h_flat, src3, dst3, att3, temp_b)

    out2d = pl.pallas_call(
        _combine_body,
        out_shape=jax.ShapeDtypeStruct((NPAD // 128, 128), f32),
    )(p0.reshape(NPAD // 128, 128),
      p1.reshape(NPAD // 128, 128),
      h_flat.reshape(NPAD // 128, 128))

    return out2d.reshape(NPAD)[:N].reshape(N, 1)
